# baseline (device time: 293045 ns/iter reference)
import jax
import jax.numpy as jnp
from jax import lax
from jax.experimental import pallas as pl
from jax.experimental.pallas import tpu as pltpu

N_DEV = 16
B_LOC = 2
SQ = 128
D = 512
H_LOC = 8
DH = 64
ROWS = B_LOC * SQ
SCALE = 0.125


def kernel(x, Wq, Wo, Wk, Wv):
    def contrib(x2d, wq, wk, wv, wo):
        q = jnp.dot(x2d, wq, preferred_element_type=jnp.float32)
        k = jnp.dot(x2d, wk, preferred_element_type=jnp.float32)
        v = jnp.dot(x2d, wv, preferred_element_type=jnp.float32)
        rows = []
        for b in range(B_LOC):
            qb = q[b * SQ:(b + 1) * SQ]
            kb = k[b * SQ:(b + 1) * SQ]
            vb = v[b * SQ:(b + 1) * SQ]
            heads = []
            for h in range(H_LOC):
                qh = qb[:, h * DH:(h + 1) * DH]
                kh = kb[:, h * DH:(h + 1) * DH]
                vh = vb[:, h * DH:(h + 1) * DH]
                s = lax.dot_general(
                    qh, kh, (((1,), (1,)), ((), ())),
                    preferred_element_type=jnp.float32,
                ) * SCALE
                m = jnp.max(s, axis=-1, keepdims=True)
                p = jnp.exp(s - m)
                l = jnp.sum(p, axis=-1, keepdims=True)
                o = jnp.dot(p, vh, preferred_element_type=jnp.float32) / l
                heads.append(o)
            rows.append(jnp.concatenate(heads, axis=1))
        att = jnp.concatenate(rows, axis=0)
        return jnp.dot(att, wo, preferred_element_type=jnp.float32)

    def body(x_ref, wq_ref, wo_ref, wk_ref, wv_ref, out_ref,
             combuf, accfin, send_sems, recv_sems, sendf_sem, recvf_sem):
        me = lax.axis_index("i")
        left = lax.rem(me + N_DEV - 1, N_DEV)
        right = lax.rem(me + 1, N_DEV)

        barrier_sem = pltpu.get_barrier_semaphore()
        for nbr in (left, right):
            pl.semaphore_signal(
                barrier_sem, inc=1,
                device_id=(nbr,), device_id_type=pl.DeviceIdType.MESH,
            )
        pl.semaphore_wait(barrier_sem, 2)

        wq = wq_ref[...]
        wk = wk_ref[...]
        wv = wv_ref[...]
        wo = wo_ref[...]

        combuf[0, 0] = x_ref[...].reshape(ROWS, D)
        combuf[0, 1] = jnp.zeros((ROWS, D), jnp.float32)

        def step(k, carry):
            c = contrib(combuf[k, 0], wq, wk, wv, wo)
            combuf[k, 1] = combuf[k, 1] + c
            rdma = pltpu.make_async_remote_copy(
                src_ref=combuf.at[k],
                dst_ref=combuf.at[k + 1],
                send_sem=send_sems.at[k],
                recv_sem=recv_sems.at[k],
                device_id=(right,),
                device_id_type=pl.DeviceIdType.MESH,
            )
            rdma.start()
            rdma.wait()
            return carry

        lax.fori_loop(0, N_DEV - 1, step, 0)

        c = contrib(combuf[N_DEV - 1, 0], wq, wk, wv, wo)
        combuf[N_DEV - 1, 1] = combuf[N_DEV - 1, 1] + c
        rdma_f = pltpu.make_async_remote_copy(
            src_ref=combuf.at[N_DEV - 1, 1],
            dst_ref=accfin,
            send_sem=sendf_sem,
            recv_sem=recvf_sem,
            device_id=(right,),
            device_id_type=pl.DeviceIdType.MESH,
        )
        rdma_f.start()
        rdma_f.wait()

        out_ref[...] = accfin[...].reshape(B_LOC, SQ, D)

    return pl.pallas_call(
        body,
        out_shape=jax.ShapeDtypeStruct((B_LOC, SQ, D), jnp.float32),
        in_specs=[pl.BlockSpec(memory_space=pltpu.VMEM)] * 5,
        out_specs=pl.BlockSpec(memory_space=pltpu.VMEM),
        scratch_shapes=[
            pltpu.VMEM((N_DEV, 2, ROWS, D), jnp.float32),
            pltpu.VMEM((ROWS, D), jnp.float32),
            pltpu.SemaphoreType.DMA((N_DEV - 1,)),
            pltpu.SemaphoreType.DMA((N_DEV - 1,)),
            pltpu.SemaphoreType.DMA,
            pltpu.SemaphoreType.DMA,
        ],
        compiler_params=pltpu.CompilerParams(collective_id=0),
    )(x, Wq, Wo, Wk, Wv)


# device time: 110052 ns/iter; 2.6628x vs baseline; 2.6628x over previous
import jax
import jax.numpy as jnp
from jax import lax
from jax.experimental import pallas as pl
from jax.experimental.pallas import tpu as pltpu

N_DEV = 16
B_LOC = 2
SQ = 128
D = 512
H_LOC = 8
DH = 64
SCALE = 0.125


def kernel(x, Wq, Wo, Wk, Wv):
    def contrib(x1, wq, wk, wv, wo):
        q = jnp.dot(x1, wq, preferred_element_type=jnp.float32)
        k = jnp.dot(x1, wk, preferred_element_type=jnp.float32)
        v = jnp.dot(x1, wv, preferred_element_type=jnp.float32)
        heads = []
        for h in range(H_LOC):
            qh = q[:, h * DH:(h + 1) * DH]
            kh = k[:, h * DH:(h + 1) * DH]
            vh = v[:, h * DH:(h + 1) * DH]
            s = lax.dot_general(
                qh, kh, (((1,), (1,)), ((), ())),
                preferred_element_type=jnp.float32,
            ) * SCALE
            m = jnp.max(s, axis=-1, keepdims=True)
            p = jnp.exp(s - m)
            l = jnp.sum(p, axis=-1, keepdims=True)
            heads.append(jnp.dot(p, vh, preferred_element_type=jnp.float32) / l)
        att = jnp.concatenate(heads, axis=1)
        return jnp.dot(att, wo, preferred_element_type=jnp.float32)

    def body(x_ref, wq_ref, wo_ref, wk_ref, wv_ref, out_ref,
             xb_r, cb_r, ar_r, fin_r,
             xb_l, cb_l, ar_l, fin_l,
             xs_sems_r, xr_sems_r, as_sems_r, arx_sems_r,
             xs_sems_l, xr_sems_l, as_sems_l, arx_sems_l,
             fs_r, fr_r, fs_l, fr_l):
        me = lax.axis_index("i")
        left = lax.rem(me + N_DEV - 1, N_DEV)
        right = lax.rem(me + 1, N_DEV)

        barrier_sem = pltpu.get_barrier_semaphore()
        for nbr in (left, right):
            pl.semaphore_signal(
                barrier_sem, inc=1,
                device_id=(nbr,), device_id_type=pl.DeviceIdType.MESH,
            )
        pl.semaphore_wait(barrier_sem, 2)

        wq = wq_ref[...]
        wk = wk_ref[...]
        wv = wv_ref[...]
        wo = wo_ref[...]

        xb_r[0] = x_ref[0]
        xb_l[0] = x_ref[1]

        def x_fwd(xb, xs_sems, xr_sems, k, dst):
            return pltpu.make_async_remote_copy(
                src_ref=xb.at[k], dst_ref=xb.at[k + 1],
                send_sem=xs_sems.at[k], recv_sem=xr_sems.at[k],
                device_id=(dst,), device_id_type=pl.DeviceIdType.MESH,
            )

        def a_fwd(cb, ar, as_sems, arx_sems, k, dst):
            return pltpu.make_async_remote_copy(
                src_ref=cb.at[k], dst_ref=ar.at[k + 1],
                send_sem=as_sems.at[k], recv_sem=arx_sems.at[k],
                device_id=(dst,), device_id_type=pl.DeviceIdType.MESH,
            )

        def ring_step(k, xb, cb, ar, xs_sems, xr_sems, as_sems, arx_sems, dst):
            @pl.when(k > 0)
            def _():
                prev_x = x_fwd(xb, xs_sems, xr_sems, k - 1, dst)
                prev_x.wait_recv()
                prev_x.wait_send()

            x_fwd(xb, xs_sems, xr_sems, k, dst).start()
            c = contrib(xb[k], wq, wk, wv, wo)

            @pl.when(k > 0)
            def _():
                prev_a = a_fwd(cb, ar, as_sems, arx_sems, k - 1, dst)
                prev_a.wait_recv()
                prev_a.wait_send()
                cb[k] = c + ar[k]

            @pl.when(k == 0)
            def _():
                cb[k] = c

            a_fwd(cb, ar, as_sems, arx_sems, k, dst).start()

        def step(k, carry):
            ring_step(k, xb_r, cb_r, ar_r, xs_sems_r, xr_sems_r,
                      as_sems_r, arx_sems_r, right)
            ring_step(k, xb_l, cb_l, ar_l, xs_sems_l, xr_sems_l,
                      as_sems_l, arx_sems_l, left)
            return carry

        lax.fori_loop(0, N_DEV - 1, step, 0)

        def ring_tail(xb, cb, ar, xs_sems, xr_sems, as_sems, arx_sems,
                      fin, fs, fr, dst):
            k = N_DEV - 1
            prev_x = x_fwd(xb, xs_sems, xr_sems, k - 1, dst)
            prev_x.wait_recv()
            prev_x.wait_send()
            c = contrib(xb[k], wq, wk, wv, wo)
            prev_a = a_fwd(cb, ar, as_sems, arx_sems, k - 1, dst)
            prev_a.wait_recv()
            prev_a.wait_send()
            cb[k] = c + ar[k]
            fin_rdma = pltpu.make_async_remote_copy(
                src_ref=cb.at[k], dst_ref=fin,
                send_sem=fs, recv_sem=fr,
                device_id=(dst,), device_id_type=pl.DeviceIdType.MESH,
            )
            fin_rdma.start()
            return fin_rdma

        fin_rdma_r = ring_tail(xb_r, cb_r, ar_r, xs_sems_r, xr_sems_r,
                               as_sems_r, arx_sems_r, fin_r, fs_r, fr_r, right)
        fin_rdma_l = ring_tail(xb_l, cb_l, ar_l, xs_sems_l, xr_sems_l,
                               as_sems_l, arx_sems_l, fin_l, fs_l, fr_l, left)
        fin_rdma_r.wait()
        fin_rdma_l.wait()

        out_ref[0] = fin_r[...]
        out_ref[1] = fin_l[...]

    ring_scratch = [
        pltpu.VMEM((N_DEV, SQ, D), jnp.float32),
        pltpu.VMEM((N_DEV, SQ, D), jnp.float32),
        pltpu.VMEM((N_DEV, SQ, D), jnp.float32),
        pltpu.VMEM((SQ, D), jnp.float32),
    ]
    ring_sems = [pltpu.SemaphoreType.DMA((N_DEV - 1,))] * 4

    return pl.pallas_call(
        body,
        out_shape=jax.ShapeDtypeStruct((B_LOC, SQ, D), jnp.float32),
        in_specs=[pl.BlockSpec(memory_space=pltpu.VMEM)] * 5,
        out_specs=pl.BlockSpec(memory_space=pltpu.VMEM),
        scratch_shapes=(
            ring_scratch + ring_scratch
            + ring_sems + ring_sems
            + [pltpu.SemaphoreType.DMA] * 4
        ),
        compiler_params=pltpu.CompilerParams(collective_id=0),
    )(x, Wq, Wo, Wk, Wv)
